# Initial kernel scaffold; baseline (speedup 1.0000x reference)
import jax
import jax.numpy as jnp
from jax.experimental import pallas as pl


def kernel(center_ids, context_ids, negative_ids, center_table, context_table):
    raise NotImplementedError("write your pallas kernel here")


# speedup vs baseline: 4.0250x; 4.0250x over previous
import functools

import jax
import jax.numpy as jnp
from jax import lax
from jax.experimental import pallas as pl
from jax.experimental.pallas import tpu as pltpu
from jax.experimental.pallas import tpu_sc as plsc

VOCAB = 1000000
DIM = 64
BATCH = 16384
NNEG = 20

NC = 2
NS = 16
NW = NC * NS
BPW = BATCH // NW
CB = 64
NCHUNK = BPW // CB
NEG_SEG = 128
NEG_PER_CHUNK = CB * NNEG
NEG_DMAS = NEG_PER_CHUNK // NEG_SEG


def _sc_scores_body(cen_ids, ctx_ids, neg_ids, cen_tab, ctx_tab, out,
                    idxc, idxx, idxn, cenv, ctxv, negv, score, sem):
  wid = lax.axis_index("s") * NC + lax.axis_index("c")

  pltpu.sync_copy(cen_ids.at[wid], idxc)
  pltpu.sync_copy(ctx_ids.at[wid], idxx)
  pltpu.sync_copy(neg_ids.at[wid], idxn)

  iota16 = lax.iota(jnp.int32, 16)

  for i in range(NCHUNK):
    copies = [
        pltpu.async_copy(cen_tab.at[idxc.at[i]], cenv, sem),
        pltpu.async_copy(ctx_tab.at[idxx.at[i]], ctxv, sem),
    ]
    for j in range(NEG_DMAS):
      copies.append(
          pltpu.async_copy(ctx_tab.at[idxn.at[i * NEG_DMAS + j]],
                           negv.at[pl.ds(j * NEG_SEG, NEG_SEG)], sem))
    for c in copies:
      c.wait()

    for g in range(CB // 16):
      rows = g * 16 + iota16
      nrows = rows * NNEG

      def body(d, accs, rows=rows, nrows=nrows):
        cold = jnp.broadcast_to(d, (16,))
        c = plsc.load_gather(cenv, [rows, cold])
        x = plsc.load_gather(ctxv, [rows, cold])
        new = [accs[0] + c * x]
        for n in range(NNEG):
          ng = plsc.load_gather(negv, [nrows + n, cold])
          new.append(accs[n + 1] + c * ng)
        return tuple(new)

      zeros = jnp.zeros((16,), jnp.float32)
      accs = lax.fori_loop(0, DIM, body, (zeros,) * (NNEG + 1))

      off = i * CB + g * 16
      score[NNEG, pl.ds(off, 16)] = accs[0]
      for n in range(NNEG):
        score[n, pl.ds(off, 16)] = -accs[n + 1]

  pltpu.sync_copy(score, out.at[wid])


_sc_scores = functools.partial(
    pl.kernel,
    out_type=jax.ShapeDtypeStruct((NW, NNEG + 1, BPW), jnp.float32),
    mesh=plsc.VectorSubcoreMesh(core_axis_name="c", subcore_axis_name="s"),
    compiler_params=pltpu.CompilerParams(
        needs_layout_passes=False, use_tc_tiling_on_sc=False),
    scratch_types=[
        pltpu.VMEM((NCHUNK, CB), jnp.int32),
        pltpu.VMEM((NCHUNK, CB), jnp.int32),
        pltpu.VMEM((NCHUNK * NEG_DMAS, NEG_SEG), jnp.int32),
        pltpu.VMEM((CB, DIM), jnp.float32),
        pltpu.VMEM((CB, DIM), jnp.float32),
        pltpu.VMEM((NEG_PER_CHUNK, DIM), jnp.float32),
        pltpu.VMEM((NNEG + 1, BPW), jnp.float32),
        pltpu.SemaphoreType.DMA,
    ],
)(_sc_scores_body)


def _loss_body(s_ref, o_ref):
  x = s_ref[...]
  y = jnp.minimum(x, 0.0) - jnp.log1p(jnp.exp(-jnp.abs(x)))
  o_ref[0, 0] = -jnp.sum(y) * (1.0 / BATCH)


_loss = pl.pallas_call(
    _loss_body,
    out_shape=jax.ShapeDtypeStruct((1, 1), jnp.float32),
    out_specs=pl.BlockSpec(memory_space=pltpu.SMEM),
)


def kernel(center_ids, context_ids, negative_ids, center_table, context_table):
  cen_ids = center_ids.astype(jnp.int32).reshape(NW, NCHUNK, CB)
  ctx_ids = context_ids.astype(jnp.int32).reshape(NW, NCHUNK, CB)
  neg_ids = negative_ids.astype(jnp.int32).reshape(
      NW, NCHUNK * NEG_DMAS, NEG_SEG)
  scores = _sc_scores(cen_ids, ctx_ids, neg_ids, center_table, context_table)
  return _loss(scores.reshape(NW * (NNEG + 1), BPW))[0, 0]


# speedup vs baseline: 4.7834x; 1.1884x over previous
import functools

import jax
import jax.numpy as jnp
from jax import lax
from jax.experimental import pallas as pl
from jax.experimental.pallas import tpu as pltpu
from jax.experimental.pallas import tpu_sc as plsc

VOCAB = 1000000
DIM = 64
BATCH = 16384
NNEG = 20

NC = 2
NS = 16
NW = NC * NS
BPW = BATCH // NW
CB = 64
NCHUNK = BPW // CB
NEG_SEG = 128
NEG_PER_CHUNK = CB * NNEG
NEG_DMAS = NEG_PER_CHUNK // NEG_SEG


def _sc_scores_body(cen_ids, ctx_ids, neg_ids, cen_tab, ctx_tab, out,
                    idxc, idxx, idxn, cenv, ctxv, negv, score, sem):
  wid = lax.axis_index("s") * NC + lax.axis_index("c")

  pltpu.sync_copy(cen_ids.at[pl.ds(wid * BPW, BPW)], idxc)
  pltpu.sync_copy(ctx_ids.at[pl.ds(wid * BPW, BPW)], idxx)
  pltpu.sync_copy(neg_ids.at[pl.ds(wid * BPW * NNEG, BPW * NNEG)], idxn)

  for i in range(NCHUNK):
    copies = [
        pltpu.async_copy(cen_tab.at[idxc.at[pl.ds(i * CB, CB)]], cenv, sem),
        pltpu.async_copy(ctx_tab.at[idxx.at[pl.ds(i * CB, CB)]], ctxv, sem),
    ]
    for j in range(NEG_DMAS):
      copies.append(
          pltpu.async_copy(
              ctx_tab.at[idxn.at[pl.ds(i * NEG_PER_CHUNK + j * NEG_SEG,
                                       NEG_SEG)]],
              negv.at[pl.ds(j * NEG_SEG, NEG_SEG)], sem))
    for c in copies:
      c.wait()

    lane15 = lax.iota(jnp.int32, 16) == 15

    def body(b, _, i=i):
      col = jnp.broadcast_to(i * CB + b, (16,))
      cs = [cenv[b, pl.ds(16 * k, 16)] for k in range(4)]
      xs = [ctxv[b, pl.ds(16 * k, 16)] for k in range(4)]
      p = cs[0] * xs[0] + cs[1] * xs[1] + cs[2] * xs[2] + cs[3] * xs[3]
      plsc.store_scatter(score, [jnp.full((16,), NNEG, jnp.int32), col],
                         plsc.cumsum(p), mask=lane15)
      for n in range(NNEG):
        r = b * NNEG + n
        gs = [negv[r, pl.ds(16 * k, 16)] for k in range(4)]
        q = cs[0] * gs[0] + cs[1] * gs[1] + cs[2] * gs[2] + cs[3] * gs[3]
        plsc.store_scatter(score, [jnp.full((16,), n, jnp.int32), col],
                           -plsc.cumsum(q), mask=lane15)
      return 0

    lax.fori_loop(0, CB, body, 0)

  pltpu.sync_copy(score, out.at[pl.ds(wid * (NNEG + 1), NNEG + 1)])


_sc_scores = functools.partial(
    pl.kernel,
    out_type=jax.ShapeDtypeStruct((NW * (NNEG + 1), BPW), jnp.float32),
    mesh=plsc.VectorSubcoreMesh(core_axis_name="c", subcore_axis_name="s"),
    compiler_params=pltpu.CompilerParams(
        needs_layout_passes=False, use_tc_tiling_on_sc=False),
    scratch_types=[
        pltpu.VMEM((BPW,), jnp.int32),
        pltpu.VMEM((BPW,), jnp.int32),
        pltpu.VMEM((BPW * NNEG,), jnp.int32),
        pltpu.VMEM((CB, DIM), jnp.float32),
        pltpu.VMEM((CB, DIM), jnp.float32),
        pltpu.VMEM((NEG_PER_CHUNK, DIM), jnp.float32),
        pltpu.VMEM((NNEG + 1, BPW), jnp.float32),
        pltpu.SemaphoreType.DMA,
    ],
)(_sc_scores_body)


def _loss_body(s_ref, o_ref):
  x = s_ref[...]
  y = jnp.minimum(x, 0.0) - jnp.log1p(jnp.exp(-jnp.abs(x)))
  o_ref[0, 0] = -jnp.sum(y) * (1.0 / BATCH)


_loss = pl.pallas_call(
    _loss_body,
    out_shape=jax.ShapeDtypeStruct((1, 1), jnp.float32),
    out_specs=pl.BlockSpec(memory_space=pltpu.SMEM),
)


def kernel(center_ids, context_ids, negative_ids, center_table, context_table):
  cen_ids = center_ids.astype(jnp.int32)
  ctx_ids = context_ids.astype(jnp.int32)
  neg_ids = negative_ids.astype(jnp.int32).reshape(BATCH * NNEG)
  scores = _sc_scores(cen_ids, ctx_ids, neg_ids, center_table, context_table)
  return _loss(scores)[0, 0]
